# baseline (speedup 1.0000x reference)
import functools

import jax
import jax.numpy as jnp
from jax import lax
from jax.experimental import pallas as pl
from jax.experimental.pallas import tpu as pltpu
from jax.experimental.pallas import tpu_sc as plsc

N = 50000
A = 8000
K = 16
L = 16
NW = 32
RADIUS = 9.0
SCALES = (1.0, 2.0, 3.0, 5.0, 10.0)
EPS = 1e-6

GP = N // L
GA = A // L
CP = 16
CA = 8

_SIZES = (36, 6, 42, 6, 256, 16, 16, 1, 128, 8, 24, 64, 8)
_OFFS = []
_o = 0
for _s in _SIZES:
    _OFFS.append(_o)
    _o += _s
(OFF_A1, OFF_BA1, OFF_A2, OFF_BA2, OFF_OR1, OFF_BOR1, OFF_OR2, OFF_BOR2,
 OFF_IN, OFF_BIN, OFF_AC, OFF_OUT, OFF_BOUT) = _OFFS
WLEN = ((_o + 7) // 8) * 8

_PARAMS = pltpu.CompilerParams(needs_layout_passes=False,
                               use_tc_tiling_on_sc=False)


def _iota():
    return lax.iota(jnp.int32, L)


def _full(v):
    return jnp.full((L,), v, jnp.int32)


def _col(ref, r0, c):
    return plsc.load_gather(ref, [_iota() + r0, _full(c)])


def _putcol(ref, r0, c, x):
    plsc.store_scatter(ref, [_iota() + r0, _full(c)], x)


def _leaky(x):
    return jnp.maximum(x, 0.2 * x)


def _rsqrt(x):
    i = plsc.bitcast(x, jnp.int32)
    y = plsc.bitcast(jnp.int32(0x5F3759DF) - (i >> 1), jnp.float32)
    for _ in range(3):
        t = (0.5 * x) * y
        y = y * (1.5 - t * y)
    return y


def _wid():
    return lax.axis_index("s") * 2 + lax.axis_index("c")


def _span(wid, total):
    q, r = divmod(total, NW)
    start = wid * q + jnp.minimum(wid, r)
    cnt = q + jnp.where(wid < r, 1, 0).astype(jnp.int32)
    return start, cnt


_MESH = plsc.VectorSubcoreMesh(core_axis_name="c", subcore_axis_name="s")


@functools.partial(
    pl.kernel,
    out_type=(jax.ShapeDtypeStruct((A, 16), jnp.float32),
              jax.ShapeDtypeStruct((N, 16), jnp.float32)),
    mesh=_MESH,
    compiler_params=_PARAMS,
    scratch_types=[
        pltpu.VMEM((WLEN, 16), jnp.float32),
        pltpu.VMEM((CA * L, 3), jnp.float32),
        pltpu.VMEM((CA * L, 6), jnp.float32),
        pltpu.VMEM((CA * L, 16), jnp.float32),
        pltpu.VMEM((CP * L, 3), jnp.float32),
        pltpu.VMEM((CP * L, 16), jnp.float32),
    ],
)
def _k1(axyz_h, afeat_h, xyz_h, wv_h, at_h, x16_h, wv, axyz, af, abuf,
        pin, pbuf):
    pltpu.sync_copy(wv_h, wv)
    wid = _wid()

    sa, ca = _span(wid, GA)
    nca = (ca + CA - 1) // CA

    def abody(ci, _):
        gb = sa + jnp.minimum(ci * CA, ca - CA)
        base = gb * L
        pltpu.sync_copy(axyz_h.at[pl.ds(base, CA * L)], axyz)
        pltpu.sync_copy(afeat_h.at[pl.ds(base, CA * L)], af)
        for gg in range(CA):
            r0 = gg * L
            f = [_col(af, r0, c) for c in range(6)]
            for c in range(3):
                _putcol(abuf, r0, c, _col(axyz, r0, c))
            for c in range(6):
                acc = wv[OFF_BA1 + c]
                for r in range(6):
                    acc = acc + f[r] * wv[OFF_A1 + r * 6 + c]
                _putcol(abuf, r0, 3 + c, _leaky(acc))
        pltpu.sync_copy(abuf, at_h.at[pl.ds(base, CA * L)])
        return 0

    lax.fori_loop(0, nca, abody, 0)

    sp, cp = _span(wid, GP)
    ncp = (cp + CP - 1) // CP

    def pbody(ci, _):
        gb = sp + jnp.minimum(ci * CP, cp - CP)
        base = gb * L
        pltpu.sync_copy(xyz_h.at[pl.ds(base, CP * L)], pin)
        for gg in range(CP):
            r0 = gg * L
            for c in range(3):
                _putcol(pbuf, r0, c, _col(pin, r0, c))
        pltpu.sync_copy(pbuf, x16_h.at[pl.ds(base, CP * L)])
        return 0

    lax.fori_loop(0, ncp, pbody, 0)


@functools.partial(
    pl.kernel,
    out_type=jax.ShapeDtypeStruct((N, 16), jnp.float32),
    mesh=_MESH,
    compiler_params=_PARAMS,
    scratch_types=[
        pltpu.VMEM((WLEN, 16), jnp.float32),
        pltpu.VMEM((L, 3), jnp.float32), pltpu.VMEM((L, 3), jnp.float32),
        pltpu.VMEM((L, 3), jnp.float32), pltpu.VMEM((L, 3), jnp.float32),
        pltpu.VMEM((L, 16), jnp.int32), pltpu.VMEM((L, 16), jnp.int32),
        pltpu.VMEM((L, 16), jnp.int32), pltpu.VMEM((L, 16), jnp.int32),
        pltpu.VMEM((128,), jnp.int32), pltpu.VMEM((128,), jnp.int32),
        pltpu.VMEM((128,), jnp.int32), pltpu.VMEM((128,), jnp.int32),
        pltpu.VMEM((128,), jnp.int32), pltpu.VMEM((128,), jnp.int32),
        pltpu.VMEM((128,), jnp.int32), pltpu.VMEM((128,), jnp.int32),
        pltpu.VMEM((128, 16), jnp.float32), pltpu.VMEM((128, 16), jnp.float32),
        pltpu.VMEM((128, 16), jnp.float32), pltpu.VMEM((128, 16), jnp.float32),
        pltpu.VMEM((128, 16), jnp.float32), pltpu.VMEM((128, 16), jnp.float32),
        pltpu.VMEM((128, 16), jnp.float32), pltpu.VMEM((128, 16), jnp.float32),
        pltpu.VMEM((L, 16), jnp.float32), pltpu.VMEM((L, 16), jnp.float32),
        pltpu.SemaphoreType.DMA, pltpu.SemaphoreType.DMA,
        pltpu.SemaphoreType.DMA, pltpu.SemaphoreType.DMA,
        pltpu.SemaphoreType.DMA, pltpu.SemaphoreType.DMA,
    ],
)
def _k2(xyz_h, nrm_h, nn_h, ai_h, x16_h, at_h, wv_h, sh_h,
        wv,
        pxyz0, pxyz1, pn0, pn1, nn0, nn1, ai0, ai1,
        ixa0, ixb0, ixc0, ixd0, ixa1, ixb1, ixc1, ixd1,
        nra0, nrb0, ara0, arb0, nra1, nrb1, ara1, arb1,
        outb0, outb1,
        semL0, semL1, semG0, semG1, semW0, semW1):
    pltpu.sync_copy(wv_h, wv)
    wid = _wid()
    start, cnt = _span(wid, GP)
    PX = (pxyz0, pxyz1)
    PN = (pn0, pn1)
    NNB = (nn0, nn1)
    AIB = (ai0, ai1)
    IX = ((ixa0, ixb0, ixc0, ixd0), (ixa1, ixb1, ixc1, ixd1))
    NR = ((nra0, nrb0), (nra1, nrb1))
    AR = ((ara0, arb0), (ara1, arb1))
    OB = (outb0, outb1)
    SL = (semL0, semL1)
    SG = (semG0, semG1)
    SW = (semW0, semW1)

    def fire_linear(s, g):
        base = (start + g) * L
        pltpu.async_copy(xyz_h.at[pl.ds(base, L)], PX[s], SL[s])
        pltpu.async_copy(nrm_h.at[pl.ds(base, L)], PN[s], SL[s])
        pltpu.async_copy(nn_h.at[pl.ds(base, L)], NNB[s], SL[s])
        pltpu.async_copy(ai_h.at[pl.ds(base, L)], AIB[s], SL[s])

    def wait_linear(s):
        for src, dst in ((xyz_h, PX[s]), (nrm_h, PN[s]),
                         (nn_h, NNB[s]), (ai_h, AIB[s])):
            pltpu.make_async_copy(src.at[pl.ds(0, L)], dst, SL[s]).wait()

    def fire_gathers(s):
        ixa, ixb, ixc, ixd = IX[s]
        for k in range(K):
            cn = _col(NNB[s], 0, k)
            cv = _col(AIB[s], 0, k)
            if k < 8:
                ixa[pl.ds(k * L, L)] = cn
                ixc[pl.ds(k * L, L)] = cv
            else:
                ixb[pl.ds((k - 8) * L, L)] = cn
                ixd[pl.ds((k - 8) * L, L)] = cv
        pltpu.async_copy(x16_h.at[ixa], NR[s][0], SG[s])
        pltpu.async_copy(x16_h.at[ixb], NR[s][1], SG[s])
        pltpu.async_copy(at_h.at[ixc], AR[s][0], SG[s])
        pltpu.async_copy(at_h.at[ixd], AR[s][1], SG[s])

    def wait_gathers(s):
        ixa, ixb, ixc, ixd = IX[s]
        pltpu.make_async_copy(x16_h.at[ixa], NR[s][0], SG[s]).wait()
        pltpu.make_async_copy(x16_h.at[ixb], NR[s][1], SG[s]).wait()
        pltpu.make_async_copy(at_h.at[ixc], AR[s][0], SG[s]).wait()
        pltpu.make_async_copy(at_h.at[ixd], AR[s][1], SG[s]).wait()

    def wait_write(s):
        pltpu.make_async_copy(OB[s], sh_h.at[pl.ds(0, L)], SW[s]).wait()

    def pvec_of(s):
        nx = _col(PN[s], 0, 0)
        ny = _col(PN[s], 0, 1)
        nz = _col(PN[s], 0, 2)
        s2 = nx * nx + ny * ny + nz * nz
        rden = 1.0 / (s2 * _rsqrt(s2) + EPS)
        return (nx * rden, ny * rden, nz * rden,
                _col(PX[s], 0, 0), _col(PX[s], 0, 1), _col(PX[s], 0, 2))

    def compute(s, pvec, g):
        nx, ny, nz, px, py, pz = pvec
        zero = jnp.full((L,), 0.0, jnp.float32)
        sw = [zero] * 5
        swp = [zero] * 5
        swp2 = [zero] * 5
        swt = [zero] * 5
        for k in range(K):
            nr = NR[s][0] if k < 8 else NR[s][1]
            r0 = (k % 8) * L
            dxx = _col(nr, r0, 0) - px
            dxy = _col(nr, r0, 1) - py
            dxz = _col(nr, r0, 2) - pz
            d2v = dxx * dxx + dxy * dxy + dxz * dxz
            pr = dxx * nx + dxy * ny + dxz * nz
            tg = jnp.maximum(d2v - pr * pr, 0.0)
            for si, sc in enumerate(SCALES):
                w = jnp.exp(d2v * (-1.0 / (2.0 * sc * sc)))
                sw[si] = sw[si] + w
                wp = w * pr
                swp[si] = swp[si] + wp
                swp2[si] = swp2[si] + wp * pr
                swt[si] = swt[si] + w * tg
        feats = []
        for si in range(5):
            rsw = 1.0 / (sw[si] + EPS)
            wp = swp[si] * rsw
            wp2 = swp2[si] * rsw
            rwt = 1.0 / (swt[si] * rsw + EPS)
            feats.append(2.0 * wp * rwt)
            feats.append(wp2 * rwt)

        chem = [zero] * 6
        for k in range(K):
            ar = AR[s][0] if k < 8 else AR[s][1]
            r0 = (k % 8) * L
            ax = _col(ar, r0, 0) - px
            ay = _col(ar, r0, 1) - py
            az = _col(ar, r0, 2) - pz
            se = ax * ax + ay * ay + az * az + EPS
            ad = se * _rsqrt(se)
            inv = 1.0 / (1.0 + ad)
            g6 = [_col(ar, r0, 3 + c) for c in range(6)]
            for c in range(6):
                acc = wv[OFF_BA2 + c] + inv * wv[OFF_A2 + 36 + c]
                for r in range(6):
                    acc = acc + g6[r] * wv[OFF_A2 + r * 6 + c]
                chem[c] = chem[c] + _leaky(acc) * inv
        for c in range(6):
            feats.append(chem[c] * (1.0 / K))

        sorv = wv[OFF_BOR2]
        for c in range(16):
            acc = wv[OFF_BOR1 + c]
            for j in range(16):
                acc = acc + feats[j] * wv[OFF_OR1 + j * 16 + c]
            sorv = sorv + _leaky(acc) * wv[OFF_OR2 + c]

        _putcol(OB[s], 0, 0, px)
        _putcol(OB[s], 0, 1, py)
        _putcol(OB[s], 0, 2, pz)
        _putcol(OB[s], 0, 3, sorv)
        for c in range(8):
            acc = wv[OFF_BIN + c]
            for j in range(16):
                acc = acc + feats[j] * wv[OFF_IN + j * 8 + c]
            _putcol(OB[s], 0, 4 + c, jnp.maximum(acc, 0.0))
        base = (start + g) * L
        pltpu.async_copy(OB[s], sh_h.at[pl.ds(base, L)], SW[s])

    fire_linear(0, 0)
    wait_linear(0)
    pv0 = pvec_of(0)
    fire_gathers(0)
    fire_linear(1, 1)
    wait_linear(1)
    pv1 = pvec_of(1)
    fire_gathers(1)
    fire_linear(0, 2)

    def unit(g, s, pva, pvb):
        s1 = 1 - s

        @pl.when(jnp.logical_and(g >= 2, g - 2 < cnt))
        def _():
            wait_write(s)

        @pl.when(g < cnt)
        def _():
            wait_gathers(s)
            compute(s, pva, g)

        @pl.when(g + 2 < cnt)
        def _():
            wait_linear(s)

        pvc = pvec_of(s)

        @pl.when(g + 2 < cnt)
        def _():
            fire_gathers(s)

        @pl.when(g + 3 < cnt)
        def _():
            fire_linear(s1, g + 3)

        return pvb, pvc

    def pair(io, carry):
        pva, pvb = carry
        g0 = io * 2
        pva, pvb = unit(g0, 0, pva, pvb)
        pva, pvb = unit(g0 + 1, 1, pva, pvb)
        return pva, pvb

    lax.fori_loop(0, (cnt + 1) // 2, pair, (pv0, pv1))
    wait_write(0)

    @pl.when(cnt % 2 == 0)
    def _():
        wait_write(1)


@functools.partial(
    pl.kernel,
    out_type=jax.ShapeDtypeStruct((N, 8), jnp.float32),
    mesh=_MESH,
    compiler_params=_PARAMS,
    scratch_types=[
        pltpu.VMEM((WLEN, 16), jnp.float32),
        pltpu.VMEM((L, 3), jnp.float32), pltpu.VMEM((L, 3), jnp.float32),
        pltpu.VMEM((L, 3), jnp.float32), pltpu.VMEM((L, 3), jnp.float32),
        pltpu.VMEM((L, 16), jnp.int32), pltpu.VMEM((L, 16), jnp.int32),
        pltpu.VMEM((128,), jnp.int32), pltpu.VMEM((128,), jnp.int32),
        pltpu.VMEM((128,), jnp.int32), pltpu.VMEM((128,), jnp.int32),
        pltpu.VMEM((128, 16), jnp.float32), pltpu.VMEM((128, 16), jnp.float32),
        pltpu.VMEM((128, 16), jnp.float32), pltpu.VMEM((128, 16), jnp.float32),
        pltpu.VMEM((K, L), jnp.float32), pltpu.VMEM((K, L), jnp.float32),
        pltpu.VMEM((K, L), jnp.float32), pltpu.VMEM((K, L), jnp.float32),
        pltpu.VMEM((L, 8), jnp.float32), pltpu.VMEM((L, 8), jnp.float32),
        pltpu.SemaphoreType.DMA, pltpu.SemaphoreType.DMA,
        pltpu.SemaphoreType.DMA, pltpu.SemaphoreType.DMA,
        pltpu.SemaphoreType.DMA, pltpu.SemaphoreType.DMA,
    ],
)
def _k3(xyz_h, nrm_h, nn_h, sh_h, wv_h, out_h,
        wv,
        pxyz0, pxyz1, pn0, pn1, nn0, nn1,
        ixa0, ixb0, ixa1, ixb1,
        nra0, nrb0, nra1, nrb1,
        bdx, bdy, bdz, bd2,
        outb0, outb1,
        semL0, semL1, semG0, semG1, semW0, semW1):
    pltpu.sync_copy(wv_h, wv)
    wid = _wid()
    start, cnt = _span(wid, GP)
    w2c = -1.0 / (2.0 * RADIUS * RADIUS)
    invr = 1.0 / RADIUS
    PX = (pxyz0, pxyz1)
    PN = (pn0, pn1)
    NNB = (nn0, nn1)
    IX = ((ixa0, ixb0), (ixa1, ixb1))
    NR = ((nra0, nrb0), (nra1, nrb1))
    OB = (outb0, outb1)
    SL = (semL0, semL1)
    SG = (semG0, semG1)
    SW = (semW0, semW1)

    def fire_linear(s, g):
        base = (start + g) * L
        pltpu.async_copy(xyz_h.at[pl.ds(base, L)], PX[s], SL[s])
        pltpu.async_copy(nrm_h.at[pl.ds(base, L)], PN[s], SL[s])
        pltpu.async_copy(nn_h.at[pl.ds(base, L)], NNB[s], SL[s])

    def wait_linear(s):
        for src, dst in ((xyz_h, PX[s]), (nrm_h, PN[s]), (nn_h, NNB[s])):
            pltpu.make_async_copy(src.at[pl.ds(0, L)], dst, SL[s]).wait()

    def fire_gathers(s):
        ixa, ixb = IX[s]
        for k in range(K):
            cn = _col(NNB[s], 0, k)
            if k < 8:
                ixa[pl.ds(k * L, L)] = cn
            else:
                ixb[pl.ds((k - 8) * L, L)] = cn
        pltpu.async_copy(sh_h.at[ixa], NR[s][0], SG[s])
        pltpu.async_copy(sh_h.at[ixb], NR[s][1], SG[s])

    def wait_gathers(s):
        ixa, ixb = IX[s]
        pltpu.make_async_copy(sh_h.at[ixa], NR[s][0], SG[s]).wait()
        pltpu.make_async_copy(sh_h.at[ixb], NR[s][1], SG[s]).wait()

    def wait_write(s):
        pltpu.make_async_copy(OB[s], out_h.at[pl.ds(0, L)], SW[s]).wait()

    def pvec_of(s):
        nx = _col(PN[s], 0, 0)
        ny = _col(PN[s], 0, 1)
        nz = _col(PN[s], 0, 2)
        s2 = nx * nx + ny * ny + nz * nz
        rden = 1.0 / (s2 * _rsqrt(s2) + EPS)
        return (nx * rden, ny * rden, nz * rden,
                _col(PX[s], 0, 0), _col(PX[s], 0, 1), _col(PX[s], 0, 2))

    def compute(s, pvec, g):
        nx, ny, nz, px, py, pz = pvec
        zero = jnp.full((L,), 0.0, jnp.float32)
        tx, ty, tz = zero, zero, zero
        for k in range(K):
            nr = NR[s][0] if k < 8 else NR[s][1]
            r0 = (k % 8) * L
            dxx = _col(nr, r0, 0) - px
            dxy = _col(nr, r0, 1) - py
            dxz = _col(nr, r0, 2) - pz
            sj = _col(nr, r0, 3)
            d2v = dxx * dxx + dxy * dxy + dxz * dxz
            bdx[k] = dxx
            bdy[k] = dxy
            bdz[k] = dxz
            bd2[k] = d2v
            pr = dxx * nx + dxy * ny + dxz * nz
            ws = jnp.exp(d2v * w2c) * sj
            tx = tx + ws * (dxx - pr * nx)
            ty = ty + ws * (dxy - pr * ny)
            tz = tz + ws * (dxz - pr * nz)
        s2t = tx * tx + ty * ty + tz * tz
        rdt = 1.0 / (s2t * _rsqrt(s2t) + EPS)
        ux, uy, uz = tx * rdt, ty * rdt, tz * rdt
        vx = ny * uz - nz * uy
        vy = nz * ux - nx * uz
        vz = nx * uy - ny * ux

        agg = [zero] * 8
        for k in range(K):
            nr = NR[s][0] if k < 8 else NR[s][1]
            r0 = (k % 8) * L
            dxx = bdx[k]
            dxy = bdy[k]
            dxz = bdz[k]
            d2v = bd2[k]
            pr = dxx * nx + dxy * ny + dxz * nz
            w = jnp.exp(d2v * w2c)
            xcu = (dxx * ux + dxy * uy + dxz * uz) * invr
            xcv = (dxx * vx + dxy * vy + dxz * vz) * invr
            xcn = pr * invr
            for c in range(8):
                hj = _col(nr, r0, 4 + c)
                filt = (1.0 + xcu * wv[OFF_AC + c]
                        + xcv * wv[OFF_AC + 8 + c]
                        + xcn * wv[OFF_AC + 16 + c])
                agg[c] = agg[c] + w * filt * hj
        for c in range(8):
            acc = wv[OFF_BOUT + c]
            for r in range(8):
                acc = acc + agg[r] * wv[OFF_OUT + r * 8 + c]
            plsc.store_scatter(OB[s], [_iota(), _full(c)], acc)
        base = (start + g) * L
        pltpu.async_copy(OB[s], out_h.at[pl.ds(base, L)], SW[s])

    fire_linear(0, 0)
    wait_linear(0)
    pv0 = pvec_of(0)
    fire_gathers(0)
    fire_linear(1, 1)
    wait_linear(1)
    pv1 = pvec_of(1)
    fire_gathers(1)
    fire_linear(0, 2)

    def unit(g, s, pva, pvb):
        s1 = 1 - s

        @pl.when(jnp.logical_and(g >= 2, g - 2 < cnt))
        def _():
            wait_write(s)

        @pl.when(g < cnt)
        def _():
            wait_gathers(s)
            compute(s, pva, g)

        @pl.when(g + 2 < cnt)
        def _():
            wait_linear(s)

        pvc = pvec_of(s)

        @pl.when(g + 2 < cnt)
        def _():
            fire_gathers(s)

        @pl.when(g + 3 < cnt)
        def _():
            fire_linear(s1, g + 3)

        return pvb, pvc

    def pair(io, carry):
        pva, pvb = carry
        g0 = io * 2
        pva, pvb = unit(g0, 0, pva, pvb)
        pva, pvb = unit(g0 + 1, 1, pva, pvb)
        return pva, pvb

    lax.fori_loop(0, (cnt + 1) // 2, pair, (pv0, pv1))
    wait_write(0)

    @pl.when(cnt % 2 == 0)
    def _():
        wait_write(1)


def kernel(xyz, normals, batch, atom_xyz, atom_features, batch_atoms,
           nn_idx, atom_idx,
           W_or1, b_or1, W_or2, b_or2,
           W_a1, b_a1, W_a2, b_a2,
           W_in, b_in, A_coef, W_out, b_out):
    nn_idx = nn_idx.astype(jnp.int32)
    atom_idx = atom_idx.astype(jnp.int32)
    wv = jnp.concatenate([
        W_a1.reshape(-1), b_a1,
        W_a2.reshape(-1), b_a2,
        W_or1.reshape(-1), b_or1,
        W_or2.reshape(-1), b_or2,
        W_in.reshape(-1), b_in,
        A_coef.reshape(-1),
        W_out.reshape(-1), b_out,
    ]).astype(jnp.float32)
    wv = jnp.pad(wv, (0, WLEN - wv.shape[0]))
    wb = jnp.tile(wv[:, None], (1, 16))
    at, x16 = _k1(atom_xyz, atom_features, xyz, wb)
    sh = _k2(xyz, normals, nn_idx, atom_idx, x16, at, wb)
    return _k3(xyz, normals, nn_idx, sh, wb)


# speedup vs baseline: 1.0088x; 1.0088x over previous
import functools

import jax
import jax.numpy as jnp
from jax import lax
from jax.experimental import pallas as pl
from jax.experimental.pallas import tpu as pltpu
from jax.experimental.pallas import tpu_sc as plsc

N = 50000
A = 8000
K = 16
L = 16
NW = 32
RADIUS = 9.0
SCALES = (1.0, 2.0, 3.0, 5.0, 10.0)
EPS = 1e-6

GP = N // L
GA = A // L
CP = 16
CA = 8

_SIZES = (36, 6, 42, 6, 256, 16, 16, 1, 128, 8, 24, 64, 8)
_OFFS = []
_o = 0
for _s in _SIZES:
    _OFFS.append(_o)
    _o += _s
(OFF_A1, OFF_BA1, OFF_A2, OFF_BA2, OFF_OR1, OFF_BOR1, OFF_OR2, OFF_BOR2,
 OFF_IN, OFF_BIN, OFF_AC, OFF_OUT, OFF_BOUT) = _OFFS
WLEN = ((_o + 7) // 8) * 8

_PARAMS = pltpu.CompilerParams(needs_layout_passes=False,
                               use_tc_tiling_on_sc=False)


def _iota():
    return lax.iota(jnp.int32, L)


def _full(v):
    return jnp.full((L,), v, jnp.int32)


def _col(ref, r0, c):
    return plsc.load_gather(ref, [_iota() + r0, _full(c)])


def _putcol(ref, r0, c, x):
    plsc.store_scatter(ref, [_iota() + r0, _full(c)], x)


def _leaky(x):
    return jnp.maximum(x, 0.2 * x)


def _rsqrt(x):
    i = plsc.bitcast(x, jnp.int32)
    y = plsc.bitcast(jnp.int32(0x5F3759DF) - (i >> 1), jnp.float32)
    for _ in range(3):
        t = (0.5 * x) * y
        y = y * (1.5 - t * y)
    return y


def _wid():
    return lax.axis_index("s") * 2 + lax.axis_index("c")


def _span(wid, total):
    q, r = divmod(total, NW)
    start = wid * q + jnp.minimum(wid, r)
    cnt = q + jnp.where(wid < r, 1, 0).astype(jnp.int32)
    return start, cnt


_MESH = plsc.VectorSubcoreMesh(core_axis_name="c", subcore_axis_name="s")


@functools.partial(
    pl.kernel,
    out_type=(jax.ShapeDtypeStruct((A, 16), jnp.float32),
              jax.ShapeDtypeStruct((N, 16), jnp.float32)),
    mesh=_MESH,
    compiler_params=_PARAMS,
    scratch_types=[
        pltpu.VMEM((WLEN, 16), jnp.float32),
        pltpu.VMEM((CA * L, 3), jnp.float32),
        pltpu.VMEM((CA * L, 6), jnp.float32),
        pltpu.VMEM((CA * L, 16), jnp.float32),
        pltpu.VMEM((CP * L, 3), jnp.float32),
        pltpu.VMEM((CP * L, 16), jnp.float32),
    ],
)
def _k1(axyz_h, afeat_h, xyz_h, wv_h, at_h, x16_h, wv, axyz, af, abuf,
        pin, pbuf):
    pltpu.sync_copy(wv_h, wv)
    wid = _wid()

    sa, ca = _span(wid, GA)
    nca = (ca + CA - 1) // CA

    def abody(ci, _):
        gb = sa + jnp.minimum(ci * CA, ca - CA)
        base = gb * L
        pltpu.sync_copy(axyz_h.at[pl.ds(base, CA * L)], axyz)
        pltpu.sync_copy(afeat_h.at[pl.ds(base, CA * L)], af)
        for gg in range(CA):
            r0 = gg * L
            f = [_col(af, r0, c) for c in range(6)]
            for c in range(3):
                _putcol(abuf, r0, c, _col(axyz, r0, c))
            for c in range(6):
                acc = wv[OFF_BA1 + c]
                for r in range(6):
                    acc = acc + f[r] * wv[OFF_A1 + r * 6 + c]
                _putcol(abuf, r0, 3 + c, _leaky(acc))
        pltpu.sync_copy(abuf, at_h.at[pl.ds(base, CA * L)])
        return 0

    lax.fori_loop(0, nca, abody, 0)

    sp, cp = _span(wid, GP)
    ncp = (cp + CP - 1) // CP

    def pbody(ci, _):
        gb = sp + jnp.minimum(ci * CP, cp - CP)
        base = gb * L
        pltpu.sync_copy(xyz_h.at[pl.ds(base, CP * L)], pin)
        for gg in range(CP):
            r0 = gg * L
            for c in range(3):
                _putcol(pbuf, r0, c, _col(pin, r0, c))
        pltpu.sync_copy(pbuf, x16_h.at[pl.ds(base, CP * L)])
        return 0

    lax.fori_loop(0, ncp, pbody, 0)


@functools.partial(
    pl.kernel,
    out_type=jax.ShapeDtypeStruct((N, 16), jnp.float32),
    mesh=_MESH,
    compiler_params=_PARAMS,
    scratch_types=[
        pltpu.VMEM((WLEN, 16), jnp.float32),
        pltpu.VMEM((L, 3), jnp.float32), pltpu.VMEM((L, 3), jnp.float32),
        pltpu.VMEM((L, 3), jnp.float32), pltpu.VMEM((L, 3), jnp.float32),
        pltpu.VMEM((L, 16), jnp.int32), pltpu.VMEM((L, 16), jnp.int32),
        pltpu.VMEM((L, 16), jnp.int32), pltpu.VMEM((L, 16), jnp.int32),
        pltpu.VMEM((128,), jnp.int32), pltpu.VMEM((128,), jnp.int32),
        pltpu.VMEM((128,), jnp.int32), pltpu.VMEM((128,), jnp.int32),
        pltpu.VMEM((128,), jnp.int32), pltpu.VMEM((128,), jnp.int32),
        pltpu.VMEM((128,), jnp.int32), pltpu.VMEM((128,), jnp.int32),
        pltpu.VMEM((128, 16), jnp.float32), pltpu.VMEM((128, 16), jnp.float32),
        pltpu.VMEM((128, 16), jnp.float32), pltpu.VMEM((128, 16), jnp.float32),
        pltpu.VMEM((128, 16), jnp.float32), pltpu.VMEM((128, 16), jnp.float32),
        pltpu.VMEM((128, 16), jnp.float32), pltpu.VMEM((128, 16), jnp.float32),
        pltpu.VMEM((L, 16), jnp.float32), pltpu.VMEM((L, 16), jnp.float32),
        pltpu.SemaphoreType.DMA, pltpu.SemaphoreType.DMA,
        pltpu.SemaphoreType.DMA, pltpu.SemaphoreType.DMA,
        pltpu.SemaphoreType.DMA, pltpu.SemaphoreType.DMA,
    ],
)
def _k2(xyz_h, nrm_h, nn_h, ai_h, x16_h, at_h, wv_h, sh_h,
        wv,
        pxyz0, pxyz1, pn0, pn1, nn0, nn1, ai0, ai1,
        ixa0, ixb0, ixc0, ixd0, ixa1, ixb1, ixc1, ixd1,
        nra0, nrb0, ara0, arb0, nra1, nrb1, ara1, arb1,
        outb0, outb1,
        semL0, semL1, semG0, semG1, semW0, semW1):
    pltpu.sync_copy(wv_h, wv)
    wid = _wid()
    start, cnt = _span(wid, GP)
    PX = (pxyz0, pxyz1)
    PN = (pn0, pn1)
    NNB = (nn0, nn1)
    AIB = (ai0, ai1)
    IX = ((ixa0, ixb0, ixc0, ixd0), (ixa1, ixb1, ixc1, ixd1))
    NR = ((nra0, nrb0), (nra1, nrb1))
    AR = ((ara0, arb0), (ara1, arb1))
    OB = (outb0, outb1)
    SL = (semL0, semL1)
    SG = (semG0, semG1)
    SW = (semW0, semW1)

    def fire_linear(s, g):
        base = (start + g) * L
        pltpu.async_copy(xyz_h.at[pl.ds(base, L)], PX[s], SL[s])
        pltpu.async_copy(nrm_h.at[pl.ds(base, L)], PN[s], SL[s])
        pltpu.async_copy(nn_h.at[pl.ds(base, L)], NNB[s], SL[s])
        pltpu.async_copy(ai_h.at[pl.ds(base, L)], AIB[s], SL[s])

    def wait_linear(s):
        for src, dst in ((xyz_h, PX[s]), (nrm_h, PN[s]),
                         (nn_h, NNB[s]), (ai_h, AIB[s])):
            pltpu.make_async_copy(src.at[pl.ds(0, L)], dst, SL[s]).wait()

    def fire_gathers(s):
        ixa, ixb, ixc, ixd = IX[s]
        for k in range(K):
            cn = _col(NNB[s], 0, k)
            cv = _col(AIB[s], 0, k)
            if k < 8:
                ixa[pl.ds(k * L, L)] = cn
                ixc[pl.ds(k * L, L)] = cv
            else:
                ixb[pl.ds((k - 8) * L, L)] = cn
                ixd[pl.ds((k - 8) * L, L)] = cv
        pltpu.async_copy(x16_h.at[ixa], NR[s][0], SG[s])
        pltpu.async_copy(x16_h.at[ixb], NR[s][1], SG[s])
        pltpu.async_copy(at_h.at[ixc], AR[s][0], SG[s])
        pltpu.async_copy(at_h.at[ixd], AR[s][1], SG[s])

    def wait_gathers(s):
        ixa, ixb, ixc, ixd = IX[s]
        pltpu.make_async_copy(x16_h.at[ixa], NR[s][0], SG[s]).wait()
        pltpu.make_async_copy(x16_h.at[ixb], NR[s][1], SG[s]).wait()
        pltpu.make_async_copy(at_h.at[ixc], AR[s][0], SG[s]).wait()
        pltpu.make_async_copy(at_h.at[ixd], AR[s][1], SG[s]).wait()

    def wait_write(s):
        pltpu.make_async_copy(OB[s], sh_h.at[pl.ds(0, L)], SW[s]).wait()

    def pvec_of(s):
        nx = _col(PN[s], 0, 0)
        ny = _col(PN[s], 0, 1)
        nz = _col(PN[s], 0, 2)
        s2 = nx * nx + ny * ny + nz * nz
        rden = 1.0 / (s2 * _rsqrt(s2) + EPS)
        return (nx * rden, ny * rden, nz * rden,
                _col(PX[s], 0, 0), _col(PX[s], 0, 1), _col(PX[s], 0, 2))

    def compute(s, pvec, g):
        nx, ny, nz, px, py, pz = pvec
        zero = jnp.full((L,), 0.0, jnp.float32)
        sw = [zero] * 5
        swp = [zero] * 5
        swp2 = [zero] * 5
        swt = [zero] * 5
        for k in range(K):
            nr = NR[s][0] if k < 8 else NR[s][1]
            r0 = (k % 8) * L
            dxx = _col(nr, r0, 0) - px
            dxy = _col(nr, r0, 1) - py
            dxz = _col(nr, r0, 2) - pz
            d2v = dxx * dxx + dxy * dxy + dxz * dxz
            pr = dxx * nx + dxy * ny + dxz * nz
            tg = jnp.maximum(d2v - pr * pr, 0.0)
            for si, sc in enumerate(SCALES):
                w = jnp.exp(d2v * (-1.0 / (2.0 * sc * sc)))
                sw[si] = sw[si] + w
                wp = w * pr
                swp[si] = swp[si] + wp
                swp2[si] = swp2[si] + wp * pr
                swt[si] = swt[si] + w * tg
        feats = []
        for si in range(5):
            rsw = 1.0 / (sw[si] + EPS)
            wp = swp[si] * rsw
            wp2 = swp2[si] * rsw
            rwt = 1.0 / (swt[si] * rsw + EPS)
            feats.append(2.0 * wp * rwt)
            feats.append(wp2 * rwt)

        chem = [zero] * 6
        wa2 = [[wv[OFF_A2 + r * 6 + c] for r in range(6)] for c in range(6)]
        wa2i = [wv[OFF_A2 + 36 + c] for c in range(6)]
        ba2 = [wv[OFF_BA2 + c] for c in range(6)]
        for k in range(K):
            ar = AR[s][0] if k < 8 else AR[s][1]
            r0 = (k % 8) * L
            ax = _col(ar, r0, 0) - px
            ay = _col(ar, r0, 1) - py
            az = _col(ar, r0, 2) - pz
            se = ax * ax + ay * ay + az * az + EPS
            ad = se * _rsqrt(se)
            inv = 1.0 / (1.0 + ad)
            g6 = [_col(ar, r0, 3 + c) for c in range(6)]
            for c in range(6):
                acc = ba2[c] + inv * wa2i[c]
                for r in range(6):
                    acc = acc + g6[r] * wa2[c][r]
                chem[c] = chem[c] + _leaky(acc) * inv
        for c in range(6):
            feats.append(chem[c] * (1.0 / K))

        sorv = wv[OFF_BOR2]
        for c in range(16):
            acc = wv[OFF_BOR1 + c]
            for j in range(16):
                acc = acc + feats[j] * wv[OFF_OR1 + j * 16 + c]
            sorv = sorv + _leaky(acc) * wv[OFF_OR2 + c]

        _putcol(OB[s], 0, 0, px)
        _putcol(OB[s], 0, 1, py)
        _putcol(OB[s], 0, 2, pz)
        _putcol(OB[s], 0, 3, sorv)
        for c in range(8):
            acc = wv[OFF_BIN + c]
            for j in range(16):
                acc = acc + feats[j] * wv[OFF_IN + j * 8 + c]
            _putcol(OB[s], 0, 4 + c, jnp.maximum(acc, 0.0))
        base = (start + g) * L
        pltpu.async_copy(OB[s], sh_h.at[pl.ds(base, L)], SW[s])

    fire_linear(0, 0)
    wait_linear(0)
    pv0 = pvec_of(0)
    fire_gathers(0)
    fire_linear(1, 1)
    wait_linear(1)
    pv1 = pvec_of(1)
    fire_gathers(1)
    fire_linear(0, 2)

    def unit(g, s, pva, pvb):
        s1 = 1 - s

        @pl.when(jnp.logical_and(g >= 2, g - 2 < cnt))
        def _():
            wait_write(s)

        @pl.when(g < cnt)
        def _():
            wait_gathers(s)
            compute(s, pva, g)

        @pl.when(g + 2 < cnt)
        def _():
            wait_linear(s)

        pvc = pvec_of(s)

        @pl.when(g + 2 < cnt)
        def _():
            fire_gathers(s)

        @pl.when(g + 3 < cnt)
        def _():
            fire_linear(s1, g + 3)

        return pvb, pvc

    def pair(io, carry):
        pva, pvb = carry
        g0 = io * 2
        pva, pvb = unit(g0, 0, pva, pvb)
        pva, pvb = unit(g0 + 1, 1, pva, pvb)
        return pva, pvb

    lax.fori_loop(0, (cnt + 1) // 2, pair, (pv0, pv1))
    wait_write(0)

    @pl.when(cnt % 2 == 0)
    def _():
        wait_write(1)


@functools.partial(
    pl.kernel,
    out_type=jax.ShapeDtypeStruct((N, 8), jnp.float32),
    mesh=_MESH,
    compiler_params=_PARAMS,
    scratch_types=[
        pltpu.VMEM((WLEN, 16), jnp.float32),
        pltpu.VMEM((L, 3), jnp.float32), pltpu.VMEM((L, 3), jnp.float32),
        pltpu.VMEM((L, 3), jnp.float32), pltpu.VMEM((L, 3), jnp.float32),
        pltpu.VMEM((L, 16), jnp.int32), pltpu.VMEM((L, 16), jnp.int32),
        pltpu.VMEM((128,), jnp.int32), pltpu.VMEM((128,), jnp.int32),
        pltpu.VMEM((128,), jnp.int32), pltpu.VMEM((128,), jnp.int32),
        pltpu.VMEM((128, 16), jnp.float32), pltpu.VMEM((128, 16), jnp.float32),
        pltpu.VMEM((128, 16), jnp.float32), pltpu.VMEM((128, 16), jnp.float32),
        pltpu.VMEM((K, L), jnp.float32), pltpu.VMEM((K, L), jnp.float32),
        pltpu.VMEM((K, L), jnp.float32), pltpu.VMEM((K, L), jnp.float32),
        pltpu.VMEM((L, 8), jnp.float32), pltpu.VMEM((L, 8), jnp.float32),
        pltpu.SemaphoreType.DMA, pltpu.SemaphoreType.DMA,
        pltpu.SemaphoreType.DMA, pltpu.SemaphoreType.DMA,
        pltpu.SemaphoreType.DMA, pltpu.SemaphoreType.DMA,
    ],
)
def _k3(xyz_h, nrm_h, nn_h, sh_h, wv_h, out_h,
        wv,
        pxyz0, pxyz1, pn0, pn1, nn0, nn1,
        ixa0, ixb0, ixa1, ixb1,
        nra0, nrb0, nra1, nrb1,
        bdx, bdy, bdz, bd2,
        outb0, outb1,
        semL0, semL1, semG0, semG1, semW0, semW1):
    pltpu.sync_copy(wv_h, wv)
    wid = _wid()
    start, cnt = _span(wid, GP)
    w2c = -1.0 / (2.0 * RADIUS * RADIUS)
    invr = 1.0 / RADIUS
    PX = (pxyz0, pxyz1)
    PN = (pn0, pn1)
    NNB = (nn0, nn1)
    IX = ((ixa0, ixb0), (ixa1, ixb1))
    NR = ((nra0, nrb0), (nra1, nrb1))
    OB = (outb0, outb1)
    SL = (semL0, semL1)
    SG = (semG0, semG1)
    SW = (semW0, semW1)

    def fire_linear(s, g):
        base = (start + g) * L
        pltpu.async_copy(xyz_h.at[pl.ds(base, L)], PX[s], SL[s])
        pltpu.async_copy(nrm_h.at[pl.ds(base, L)], PN[s], SL[s])
        pltpu.async_copy(nn_h.at[pl.ds(base, L)], NNB[s], SL[s])

    def wait_linear(s):
        for src, dst in ((xyz_h, PX[s]), (nrm_h, PN[s]), (nn_h, NNB[s])):
            pltpu.make_async_copy(src.at[pl.ds(0, L)], dst, SL[s]).wait()

    def fire_gathers(s):
        ixa, ixb = IX[s]
        for k in range(K):
            cn = _col(NNB[s], 0, k)
            if k < 8:
                ixa[pl.ds(k * L, L)] = cn
            else:
                ixb[pl.ds((k - 8) * L, L)] = cn
        pltpu.async_copy(sh_h.at[ixa], NR[s][0], SG[s])
        pltpu.async_copy(sh_h.at[ixb], NR[s][1], SG[s])

    def wait_gathers(s):
        ixa, ixb = IX[s]
        pltpu.make_async_copy(sh_h.at[ixa], NR[s][0], SG[s]).wait()
        pltpu.make_async_copy(sh_h.at[ixb], NR[s][1], SG[s]).wait()

    def wait_write(s):
        pltpu.make_async_copy(OB[s], out_h.at[pl.ds(0, L)], SW[s]).wait()

    def pvec_of(s):
        nx = _col(PN[s], 0, 0)
        ny = _col(PN[s], 0, 1)
        nz = _col(PN[s], 0, 2)
        s2 = nx * nx + ny * ny + nz * nz
        rden = 1.0 / (s2 * _rsqrt(s2) + EPS)
        return (nx * rden, ny * rden, nz * rden,
                _col(PX[s], 0, 0), _col(PX[s], 0, 1), _col(PX[s], 0, 2))

    def compute(s, pvec, g):
        nx, ny, nz, px, py, pz = pvec
        zero = jnp.full((L,), 0.0, jnp.float32)
        tx, ty, tz = zero, zero, zero
        for k in range(K):
            nr = NR[s][0] if k < 8 else NR[s][1]
            r0 = (k % 8) * L
            dxx = _col(nr, r0, 0) - px
            dxy = _col(nr, r0, 1) - py
            dxz = _col(nr, r0, 2) - pz
            sj = _col(nr, r0, 3)
            d2v = dxx * dxx + dxy * dxy + dxz * dxz
            bdx[k] = dxx
            bdy[k] = dxy
            bdz[k] = dxz
            bd2[k] = d2v
            pr = dxx * nx + dxy * ny + dxz * nz
            ws = jnp.exp(d2v * w2c) * sj
            tx = tx + ws * (dxx - pr * nx)
            ty = ty + ws * (dxy - pr * ny)
            tz = tz + ws * (dxz - pr * nz)
        s2t = tx * tx + ty * ty + tz * tz
        rdt = 1.0 / (s2t * _rsqrt(s2t) + EPS)
        ux, uy, uz = tx * rdt, ty * rdt, tz * rdt
        vx = ny * uz - nz * uy
        vy = nz * ux - nx * uz
        vz = nx * uy - ny * ux

        agg = [zero] * 8
        acu = [wv[OFF_AC + c] for c in range(8)]
        acv = [wv[OFF_AC + 8 + c] for c in range(8)]
        acn = [wv[OFF_AC + 16 + c] for c in range(8)]
        for k in range(K):
            nr = NR[s][0] if k < 8 else NR[s][1]
            r0 = (k % 8) * L
            dxx = bdx[k]
            dxy = bdy[k]
            dxz = bdz[k]
            d2v = bd2[k]
            pr = dxx * nx + dxy * ny + dxz * nz
            w = jnp.exp(d2v * w2c)
            xcu = (dxx * ux + dxy * uy + dxz * uz) * invr
            xcv = (dxx * vx + dxy * vy + dxz * vz) * invr
            xcn = pr * invr
            for c in range(8):
                hj = _col(nr, r0, 4 + c)
                filt = (1.0 + xcu * acu[c]
                        + xcv * acv[c]
                        + xcn * acn[c])
                agg[c] = agg[c] + w * filt * hj
        for c in range(8):
            acc = wv[OFF_BOUT + c]
            for r in range(8):
                acc = acc + agg[r] * wv[OFF_OUT + r * 8 + c]
            plsc.store_scatter(OB[s], [_iota(), _full(c)], acc)
        base = (start + g) * L
        pltpu.async_copy(OB[s], out_h.at[pl.ds(base, L)], SW[s])

    fire_linear(0, 0)
    wait_linear(0)
    pv0 = pvec_of(0)
    fire_gathers(0)
    fire_linear(1, 1)
    wait_linear(1)
    pv1 = pvec_of(1)
    fire_gathers(1)
    fire_linear(0, 2)

    def unit(g, s, pva, pvb):
        s1 = 1 - s

        @pl.when(jnp.logical_and(g >= 2, g - 2 < cnt))
        def _():
            wait_write(s)

        @pl.when(g < cnt)
        def _():
            wait_gathers(s)
            compute(s, pva, g)

        @pl.when(g + 2 < cnt)
        def _():
            wait_linear(s)

        pvc = pvec_of(s)

        @pl.when(g + 2 < cnt)
        def _():
            fire_gathers(s)

        @pl.when(g + 3 < cnt)
        def _():
            fire_linear(s1, g + 3)

        return pvb, pvc

    def pair(io, carry):
        pva, pvb = carry
        g0 = io * 2
        pva, pvb = unit(g0, 0, pva, pvb)
        pva, pvb = unit(g0 + 1, 1, pva, pvb)
        return pva, pvb

    lax.fori_loop(0, (cnt + 1) // 2, pair, (pv0, pv1))
    wait_write(0)

    @pl.when(cnt % 2 == 0)
    def _():
        wait_write(1)


def kernel(xyz, normals, batch, atom_xyz, atom_features, batch_atoms,
           nn_idx, atom_idx,
           W_or1, b_or1, W_or2, b_or2,
           W_a1, b_a1, W_a2, b_a2,
           W_in, b_in, A_coef, W_out, b_out):
    nn_idx = nn_idx.astype(jnp.int32)
    atom_idx = atom_idx.astype(jnp.int32)
    wv = jnp.concatenate([
        W_a1.reshape(-1), b_a1,
        W_a2.reshape(-1), b_a2,
        W_or1.reshape(-1), b_or1,
        W_or2.reshape(-1), b_or2,
        W_in.reshape(-1), b_in,
        A_coef.reshape(-1),
        W_out.reshape(-1), b_out,
    ]).astype(jnp.float32)
    wv = jnp.pad(wv, (0, WLEN - wv.shape[0]))
    wb = jnp.tile(wv[:, None], (1, 16))
    at, x16 = _k1(atom_xyz, atom_features, xyz, wb)
    sh = _k2(xyz, normals, nn_idx, atom_idx, x16, at, wb)
    return _k3(xyz, normals, nn_idx, sh, wb)


# speedup vs baseline: 1.0313x; 1.0223x over previous
import functools

import jax
import jax.numpy as jnp
from jax import lax
from jax.experimental import pallas as pl
from jax.experimental.pallas import tpu as pltpu
from jax.experimental.pallas import tpu_sc as plsc

N = 50000
A = 8000
K = 16
L = 16
NW = 32
RADIUS = 9.0
SCALES = (1.0, 2.0, 3.0, 5.0, 10.0)
EPS = 1e-6

GP = N // L
GA = A // L
CP = 16
CA = 8

_SIZES = (36, 6, 42, 6, 256, 16, 16, 1, 128, 8, 24, 64, 8)
_OFFS = []
_o = 0
for _s in _SIZES:
    _OFFS.append(_o)
    _o += _s
(OFF_A1, OFF_BA1, OFF_A2, OFF_BA2, OFF_OR1, OFF_BOR1, OFF_OR2, OFF_BOR2,
 OFF_IN, OFF_BIN, OFF_AC, OFF_OUT, OFF_BOUT) = _OFFS
WLEN = ((_o + 7) // 8) * 8

_PARAMS = pltpu.CompilerParams(needs_layout_passes=False,
                               use_tc_tiling_on_sc=False)


def _iota():
    return lax.iota(jnp.int32, L)


def _full(v):
    return jnp.full((L,), v, jnp.int32)


def _col(ref, r0, c):
    return plsc.load_gather(ref, [_iota() + r0, _full(c)])


def _putcol(ref, r0, c, x):
    plsc.store_scatter(ref, [_iota() + r0, _full(c)], x)


def _leaky(x):
    return jnp.maximum(x, 0.2 * x)


def _rsqrt(x):
    i = plsc.bitcast(x, jnp.int32)
    y = plsc.bitcast(jnp.int32(0x5F3759DF) - (i >> 1), jnp.float32)
    for _ in range(3):
        t = (0.5 * x) * y
        y = y * (1.5 - t * y)
    return y


def _wid():
    return lax.axis_index("s") * 2 + lax.axis_index("c")


def _span(wid, total):
    q, r = divmod(total, NW)
    start = wid * q + jnp.minimum(wid, r)
    cnt = q + jnp.where(wid < r, 1, 0).astype(jnp.int32)
    return start, cnt


_MESH = plsc.VectorSubcoreMesh(core_axis_name="c", subcore_axis_name="s")


@functools.partial(
    pl.kernel,
    out_type=(jax.ShapeDtypeStruct((A, 16), jnp.float32),
              jax.ShapeDtypeStruct((N, 8), jnp.float32)),
    mesh=_MESH,
    compiler_params=_PARAMS,
    scratch_types=[
        pltpu.VMEM((WLEN, 16), jnp.float32),
        pltpu.VMEM((CA * L, 3), jnp.float32),
        pltpu.VMEM((CA * L, 6), jnp.float32),
        pltpu.VMEM((CA * L, 16), jnp.float32),
        pltpu.VMEM((CP * L, 3), jnp.float32),
        pltpu.VMEM((CP * L, 8), jnp.float32),
    ],
)
def _k1(axyz_h, afeat_h, xyz_h, wv_h, at_h, x16_h, wv, axyz, af, abuf,
        pin, pbuf):
    pltpu.sync_copy(wv_h, wv)
    wid = _wid()

    sa, ca = _span(wid, GA)
    nca = (ca + CA - 1) // CA

    def abody(ci, _):
        gb = sa + jnp.minimum(ci * CA, ca - CA)
        base = gb * L
        pltpu.sync_copy(axyz_h.at[pl.ds(base, CA * L)], axyz)
        pltpu.sync_copy(afeat_h.at[pl.ds(base, CA * L)], af)
        for gg in range(CA):
            r0 = gg * L
            f = [_col(af, r0, c) for c in range(6)]
            for c in range(3):
                _putcol(abuf, r0, c, _col(axyz, r0, c))
            for c in range(6):
                acc = wv[OFF_BA1 + c]
                for r in range(6):
                    acc = acc + f[r] * wv[OFF_A1 + r * 6 + c]
                _putcol(abuf, r0, 3 + c, _leaky(acc))
        pltpu.sync_copy(abuf, at_h.at[pl.ds(base, CA * L)])
        return 0

    lax.fori_loop(0, nca, abody, 0)

    sp, cp = _span(wid, GP)
    ncp = (cp + CP - 1) // CP

    def pbody(ci, _):
        gb = sp + jnp.minimum(ci * CP, cp - CP)
        base = gb * L
        pltpu.sync_copy(xyz_h.at[pl.ds(base, CP * L)], pin)
        for gg in range(CP):
            r0 = gg * L
            for c in range(3):
                _putcol(pbuf, r0, c, _col(pin, r0, c))
        pltpu.sync_copy(pbuf, x16_h.at[pl.ds(base, CP * L)])
        return 0

    lax.fori_loop(0, ncp, pbody, 0)


@functools.partial(
    pl.kernel,
    out_type=jax.ShapeDtypeStruct((N, 16), jnp.float32),
    mesh=_MESH,
    compiler_params=_PARAMS,
    scratch_types=[
        pltpu.VMEM((WLEN, 16), jnp.float32),
        pltpu.VMEM((L, 3), jnp.float32), pltpu.VMEM((L, 3), jnp.float32),
        pltpu.VMEM((L, 3), jnp.float32), pltpu.VMEM((L, 3), jnp.float32),
        pltpu.VMEM((L, 16), jnp.int32), pltpu.VMEM((L, 16), jnp.int32),
        pltpu.VMEM((L, 16), jnp.int32), pltpu.VMEM((L, 16), jnp.int32),
        pltpu.VMEM((128,), jnp.int32), pltpu.VMEM((128,), jnp.int32),
        pltpu.VMEM((128,), jnp.int32), pltpu.VMEM((128,), jnp.int32),
        pltpu.VMEM((128,), jnp.int32), pltpu.VMEM((128,), jnp.int32),
        pltpu.VMEM((128,), jnp.int32), pltpu.VMEM((128,), jnp.int32),
        pltpu.VMEM((128, 8), jnp.float32), pltpu.VMEM((128, 8), jnp.float32),
        pltpu.VMEM((128, 16), jnp.float32), pltpu.VMEM((128, 16), jnp.float32),
        pltpu.VMEM((128, 8), jnp.float32), pltpu.VMEM((128, 8), jnp.float32),
        pltpu.VMEM((128, 16), jnp.float32), pltpu.VMEM((128, 16), jnp.float32),
        pltpu.VMEM((L, 16), jnp.float32), pltpu.VMEM((L, 16), jnp.float32),
        pltpu.SemaphoreType.DMA, pltpu.SemaphoreType.DMA,
        pltpu.SemaphoreType.DMA, pltpu.SemaphoreType.DMA,
        pltpu.SemaphoreType.DMA, pltpu.SemaphoreType.DMA,
    ],
)
def _k2(xyz_h, nrm_h, nn_h, ai_h, x16_h, at_h, wv_h, sh_h,
        wv,
        pxyz0, pxyz1, pn0, pn1, nn0, nn1, ai0, ai1,
        ixa0, ixb0, ixc0, ixd0, ixa1, ixb1, ixc1, ixd1,
        nra0, nrb0, ara0, arb0, nra1, nrb1, ara1, arb1,
        outb0, outb1,
        semL0, semL1, semG0, semG1, semW0, semW1):
    pltpu.sync_copy(wv_h, wv)
    wid = _wid()
    start, cnt = _span(wid, GP)
    PX = (pxyz0, pxyz1)
    PN = (pn0, pn1)
    NNB = (nn0, nn1)
    AIB = (ai0, ai1)
    IX = ((ixa0, ixb0, ixc0, ixd0), (ixa1, ixb1, ixc1, ixd1))
    NR = ((nra0, nrb0), (nra1, nrb1))
    AR = ((ara0, arb0), (ara1, arb1))
    OB = (outb0, outb1)
    SL = (semL0, semL1)
    SG = (semG0, semG1)
    SW = (semW0, semW1)

    def fire_linear(s, g):
        base = (start + g) * L
        pltpu.async_copy(xyz_h.at[pl.ds(base, L)], PX[s], SL[s])
        pltpu.async_copy(nrm_h.at[pl.ds(base, L)], PN[s], SL[s])
        pltpu.async_copy(nn_h.at[pl.ds(base, L)], NNB[s], SL[s])
        pltpu.async_copy(ai_h.at[pl.ds(base, L)], AIB[s], SL[s])

    def wait_linear(s):
        for src, dst in ((xyz_h, PX[s]), (nrm_h, PN[s]),
                         (nn_h, NNB[s]), (ai_h, AIB[s])):
            pltpu.make_async_copy(src.at[pl.ds(0, L)], dst, SL[s]).wait()

    def fire_gathers(s):
        ixa, ixb, ixc, ixd = IX[s]
        for k in range(K):
            cn = _col(NNB[s], 0, k)
            cv = _col(AIB[s], 0, k)
            if k < 8:
                ixa[pl.ds(k * L, L)] = cn
                ixc[pl.ds(k * L, L)] = cv
            else:
                ixb[pl.ds((k - 8) * L, L)] = cn
                ixd[pl.ds((k - 8) * L, L)] = cv
        pltpu.async_copy(x16_h.at[ixa], NR[s][0], SG[s])
        pltpu.async_copy(x16_h.at[ixb], NR[s][1], SG[s])
        pltpu.async_copy(at_h.at[ixc], AR[s][0], SG[s])
        pltpu.async_copy(at_h.at[ixd], AR[s][1], SG[s])

    def wait_gathers(s):
        ixa, ixb, ixc, ixd = IX[s]
        pltpu.make_async_copy(x16_h.at[ixa], NR[s][0], SG[s]).wait()
        pltpu.make_async_copy(x16_h.at[ixb], NR[s][1], SG[s]).wait()
        pltpu.make_async_copy(at_h.at[ixc], AR[s][0], SG[s]).wait()
        pltpu.make_async_copy(at_h.at[ixd], AR[s][1], SG[s]).wait()

    def wait_write(s):
        pltpu.make_async_copy(OB[s], sh_h.at[pl.ds(0, L)], SW[s]).wait()

    def pvec_of(s):
        nx = _col(PN[s], 0, 0)
        ny = _col(PN[s], 0, 1)
        nz = _col(PN[s], 0, 2)
        s2 = nx * nx + ny * ny + nz * nz
        rden = 1.0 / (s2 * _rsqrt(s2) + EPS)
        return (nx * rden, ny * rden, nz * rden,
                _col(PX[s], 0, 0), _col(PX[s], 0, 1), _col(PX[s], 0, 2))

    def compute(s, pvec, g):
        nx, ny, nz, px, py, pz = pvec
        zero = jnp.full((L,), 0.0, jnp.float32)
        sw = [zero] * 5
        swp = [zero] * 5
        swp2 = [zero] * 5
        swt = [zero] * 5
        for k in range(K):
            nr = NR[s][0] if k < 8 else NR[s][1]
            r0 = (k % 8) * L
            dxx = _col(nr, r0, 0) - px
            dxy = _col(nr, r0, 1) - py
            dxz = _col(nr, r0, 2) - pz
            d2v = dxx * dxx + dxy * dxy + dxz * dxz
            pr = dxx * nx + dxy * ny + dxz * nz
            tg = jnp.maximum(d2v - pr * pr, 0.0)
            for si, sc in enumerate(SCALES):
                w = jnp.exp(d2v * (-1.0 / (2.0 * sc * sc)))
                sw[si] = sw[si] + w
                wp = w * pr
                swp[si] = swp[si] + wp
                swp2[si] = swp2[si] + wp * pr
                swt[si] = swt[si] + w * tg
        feats = []
        for si in range(5):
            rsw = 1.0 / (sw[si] + EPS)
            wp = swp[si] * rsw
            wp2 = swp2[si] * rsw
            rwt = 1.0 / (swt[si] * rsw + EPS)
            feats.append(2.0 * wp * rwt)
            feats.append(wp2 * rwt)

        chem = [zero] * 6
        wa2 = [[wv[OFF_A2 + r * 6 + c] for r in range(6)] for c in range(6)]
        wa2i = [wv[OFF_A2 + 36 + c] for c in range(6)]
        ba2 = [wv[OFF_BA2 + c] for c in range(6)]
        for k in range(K):
            ar = AR[s][0] if k < 8 else AR[s][1]
            r0 = (k % 8) * L
            ax = _col(ar, r0, 0) - px
            ay = _col(ar, r0, 1) - py
            az = _col(ar, r0, 2) - pz
            se = ax * ax + ay * ay + az * az + EPS
            ad = se * _rsqrt(se)
            inv = 1.0 / (1.0 + ad)
            g6 = [_col(ar, r0, 3 + c) for c in range(6)]
            for c in range(6):
                acc = ba2[c] + inv * wa2i[c]
                for r in range(6):
                    acc = acc + g6[r] * wa2[c][r]
                chem[c] = chem[c] + _leaky(acc) * inv
        for c in range(6):
            feats.append(chem[c] * (1.0 / K))

        sorv = wv[OFF_BOR2]
        for c in range(16):
            acc = wv[OFF_BOR1 + c]
            for j in range(16):
                acc = acc + feats[j] * wv[OFF_OR1 + j * 16 + c]
            sorv = sorv + _leaky(acc) * wv[OFF_OR2 + c]

        _putcol(OB[s], 0, 0, px)
        _putcol(OB[s], 0, 1, py)
        _putcol(OB[s], 0, 2, pz)
        _putcol(OB[s], 0, 3, sorv)
        for c in range(8):
            acc = wv[OFF_BIN + c]
            for j in range(16):
                acc = acc + feats[j] * wv[OFF_IN + j * 8 + c]
            _putcol(OB[s], 0, 4 + c, jnp.maximum(acc, 0.0))
        base = (start + g) * L
        pltpu.async_copy(OB[s], sh_h.at[pl.ds(base, L)], SW[s])

    fire_linear(0, 0)
    wait_linear(0)
    pv0 = pvec_of(0)
    fire_gathers(0)
    fire_linear(1, 1)
    wait_linear(1)
    pv1 = pvec_of(1)
    fire_gathers(1)
    fire_linear(0, 2)

    def unit(g, s, pva, pvb):
        s1 = 1 - s

        @pl.when(jnp.logical_and(g >= 2, g - 2 < cnt))
        def _():
            wait_write(s)

        @pl.when(g < cnt)
        def _():
            wait_gathers(s)
            compute(s, pva, g)

        @pl.when(g + 2 < cnt)
        def _():
            wait_linear(s)

        pvc = pvec_of(s)

        @pl.when(g + 2 < cnt)
        def _():
            fire_gathers(s)

        @pl.when(g + 3 < cnt)
        def _():
            fire_linear(s1, g + 3)

        return pvb, pvc

    def pair(io, carry):
        pva, pvb = carry
        g0 = io * 2
        pva, pvb = unit(g0, 0, pva, pvb)
        pva, pvb = unit(g0 + 1, 1, pva, pvb)
        return pva, pvb

    lax.fori_loop(0, (cnt + 1) // 2, pair, (pv0, pv1))
    wait_write(0)

    @pl.when(cnt % 2 == 0)
    def _():
        wait_write(1)


@functools.partial(
    pl.kernel,
    out_type=jax.ShapeDtypeStruct((N, 8), jnp.float32),
    mesh=_MESH,
    compiler_params=_PARAMS,
    scratch_types=[
        pltpu.VMEM((WLEN, 16), jnp.float32),
        pltpu.VMEM((L, 3), jnp.float32), pltpu.VMEM((L, 3), jnp.float32),
        pltpu.VMEM((L, 3), jnp.float32), pltpu.VMEM((L, 3), jnp.float32),
        pltpu.VMEM((L, 16), jnp.int32), pltpu.VMEM((L, 16), jnp.int32),
        pltpu.VMEM((128,), jnp.int32), pltpu.VMEM((128,), jnp.int32),
        pltpu.VMEM((128,), jnp.int32), pltpu.VMEM((128,), jnp.int32),
        pltpu.VMEM((128, 16), jnp.float32), pltpu.VMEM((128, 16), jnp.float32),
        pltpu.VMEM((128, 16), jnp.float32), pltpu.VMEM((128, 16), jnp.float32),
        pltpu.VMEM((K, L), jnp.float32), pltpu.VMEM((K, L), jnp.float32),
        pltpu.VMEM((K, L), jnp.float32), pltpu.VMEM((K, L), jnp.float32),
        pltpu.VMEM((L, 8), jnp.float32), pltpu.VMEM((L, 8), jnp.float32),
        pltpu.SemaphoreType.DMA, pltpu.SemaphoreType.DMA,
        pltpu.SemaphoreType.DMA, pltpu.SemaphoreType.DMA,
        pltpu.SemaphoreType.DMA, pltpu.SemaphoreType.DMA,
    ],
)
def _k3(xyz_h, nrm_h, nn_h, sh_h, wv_h, out_h,
        wv,
        pxyz0, pxyz1, pn0, pn1, nn0, nn1,
        ixa0, ixb0, ixa1, ixb1,
        nra0, nrb0, nra1, nrb1,
        bdx, bdy, bdz, bd2,
        outb0, outb1,
        semL0, semL1, semG0, semG1, semW0, semW1):
    pltpu.sync_copy(wv_h, wv)
    wid = _wid()
    start, cnt = _span(wid, GP)
    w2c = -1.0 / (2.0 * RADIUS * RADIUS)
    invr = 1.0 / RADIUS
    PX = (pxyz0, pxyz1)
    PN = (pn0, pn1)
    NNB = (nn0, nn1)
    IX = ((ixa0, ixb0), (ixa1, ixb1))
    NR = ((nra0, nrb0), (nra1, nrb1))
    OB = (outb0, outb1)
    SL = (semL0, semL1)
    SG = (semG0, semG1)
    SW = (semW0, semW1)

    def fire_linear(s, g):
        base = (start + g) * L
        pltpu.async_copy(xyz_h.at[pl.ds(base, L)], PX[s], SL[s])
        pltpu.async_copy(nrm_h.at[pl.ds(base, L)], PN[s], SL[s])
        pltpu.async_copy(nn_h.at[pl.ds(base, L)], NNB[s], SL[s])

    def wait_linear(s):
        for src, dst in ((xyz_h, PX[s]), (nrm_h, PN[s]), (nn_h, NNB[s])):
            pltpu.make_async_copy(src.at[pl.ds(0, L)], dst, SL[s]).wait()

    def fire_gathers(s):
        ixa, ixb = IX[s]
        for k in range(K):
            cn = _col(NNB[s], 0, k)
            if k < 8:
                ixa[pl.ds(k * L, L)] = cn
            else:
                ixb[pl.ds((k - 8) * L, L)] = cn
        pltpu.async_copy(sh_h.at[ixa], NR[s][0], SG[s])
        pltpu.async_copy(sh_h.at[ixb], NR[s][1], SG[s])

    def wait_gathers(s):
        ixa, ixb = IX[s]
        pltpu.make_async_copy(sh_h.at[ixa], NR[s][0], SG[s]).wait()
        pltpu.make_async_copy(sh_h.at[ixb], NR[s][1], SG[s]).wait()

    def wait_write(s):
        pltpu.make_async_copy(OB[s], out_h.at[pl.ds(0, L)], SW[s]).wait()

    def pvec_of(s):
        nx = _col(PN[s], 0, 0)
        ny = _col(PN[s], 0, 1)
        nz = _col(PN[s], 0, 2)
        s2 = nx * nx + ny * ny + nz * nz
        rden = 1.0 / (s2 * _rsqrt(s2) + EPS)
        return (nx * rden, ny * rden, nz * rden,
                _col(PX[s], 0, 0), _col(PX[s], 0, 1), _col(PX[s], 0, 2))

    def compute(s, pvec, g):
        nx, ny, nz, px, py, pz = pvec
        zero = jnp.full((L,), 0.0, jnp.float32)
        tx, ty, tz = zero, zero, zero
        for k in range(K):
            nr = NR[s][0] if k < 8 else NR[s][1]
            r0 = (k % 8) * L
            dxx = _col(nr, r0, 0) - px
            dxy = _col(nr, r0, 1) - py
            dxz = _col(nr, r0, 2) - pz
            sj = _col(nr, r0, 3)
            d2v = dxx * dxx + dxy * dxy + dxz * dxz
            bdx[k] = dxx
            bdy[k] = dxy
            bdz[k] = dxz
            bd2[k] = d2v
            pr = dxx * nx + dxy * ny + dxz * nz
            ws = jnp.exp(d2v * w2c) * sj
            tx = tx + ws * (dxx - pr * nx)
            ty = ty + ws * (dxy - pr * ny)
            tz = tz + ws * (dxz - pr * nz)
        s2t = tx * tx + ty * ty + tz * tz
        rdt = 1.0 / (s2t * _rsqrt(s2t) + EPS)
        ux, uy, uz = tx * rdt, ty * rdt, tz * rdt
        vx = ny * uz - nz * uy
        vy = nz * ux - nx * uz
        vz = nx * uy - ny * ux

        agg = [zero] * 8
        acu = [wv[OFF_AC + c] for c in range(8)]
        acv = [wv[OFF_AC + 8 + c] for c in range(8)]
        acn = [wv[OFF_AC + 16 + c] for c in range(8)]
        for k in range(K):
            nr = NR[s][0] if k < 8 else NR[s][1]
            r0 = (k % 8) * L
            dxx = bdx[k]
            dxy = bdy[k]
            dxz = bdz[k]
            d2v = bd2[k]
            pr = dxx * nx + dxy * ny + dxz * nz
            w = jnp.exp(d2v * w2c)
            xcu = (dxx * ux + dxy * uy + dxz * uz) * invr
            xcv = (dxx * vx + dxy * vy + dxz * vz) * invr
            xcn = pr * invr
            for c in range(8):
                hj = _col(nr, r0, 4 + c)
                filt = (1.0 + xcu * acu[c]
                        + xcv * acv[c]
                        + xcn * acn[c])
                agg[c] = agg[c] + w * filt * hj
        for c in range(8):
            acc = wv[OFF_BOUT + c]
            for r in range(8):
                acc = acc + agg[r] * wv[OFF_OUT + r * 8 + c]
            plsc.store_scatter(OB[s], [_iota(), _full(c)], acc)
        base = (start + g) * L
        pltpu.async_copy(OB[s], out_h.at[pl.ds(base, L)], SW[s])

    fire_linear(0, 0)
    wait_linear(0)
    pv0 = pvec_of(0)
    fire_gathers(0)
    fire_linear(1, 1)
    wait_linear(1)
    pv1 = pvec_of(1)
    fire_gathers(1)
    fire_linear(0, 2)

    def unit(g, s, pva, pvb):
        s1 = 1 - s

        @pl.when(jnp.logical_and(g >= 2, g - 2 < cnt))
        def _():
            wait_write(s)

        @pl.when(g < cnt)
        def _():
            wait_gathers(s)
            compute(s, pva, g)

        @pl.when(g + 2 < cnt)
        def _():
            wait_linear(s)

        pvc = pvec_of(s)

        @pl.when(g + 2 < cnt)
        def _():
            fire_gathers(s)

        @pl.when(g + 3 < cnt)
        def _():
            fire_linear(s1, g + 3)

        return pvb, pvc

    def pair(io, carry):
        pva, pvb = carry
        g0 = io * 2
        pva, pvb = unit(g0, 0, pva, pvb)
        pva, pvb = unit(g0 + 1, 1, pva, pvb)
        return pva, pvb

    lax.fori_loop(0, (cnt + 1) // 2, pair, (pv0, pv1))
    wait_write(0)

    @pl.when(cnt % 2 == 0)
    def _():
        wait_write(1)


def kernel(xyz, normals, batch, atom_xyz, atom_features, batch_atoms,
           nn_idx, atom_idx,
           W_or1, b_or1, W_or2, b_or2,
           W_a1, b_a1, W_a2, b_a2,
           W_in, b_in, A_coef, W_out, b_out):
    nn_idx = nn_idx.astype(jnp.int32)
    atom_idx = atom_idx.astype(jnp.int32)
    wv = jnp.concatenate([
        W_a1.reshape(-1), b_a1,
        W_a2.reshape(-1), b_a2,
        W_or1.reshape(-1), b_or1,
        W_or2.reshape(-1), b_or2,
        W_in.reshape(-1), b_in,
        A_coef.reshape(-1),
        W_out.reshape(-1), b_out,
    ]).astype(jnp.float32)
    wv = jnp.pad(wv, (0, WLEN - wv.shape[0]))
    wb = jnp.tile(wv[:, None], (1, 16))
    at, x16 = _k1(atom_xyz, atom_features, xyz, wb)
    sh = _k2(xyz, normals, nn_idx, atom_idx, x16, at, wb)
    return _k3(xyz, normals, nn_idx, sh, wb)
